# Initial kernel scaffold; baseline (speedup 1.0000x reference)
import jax
import jax.numpy as jnp
from jax.experimental import pallas as pl


def kernel(x, target):
    raise NotImplementedError("write your pallas kernel here")


# speedup vs baseline: 2.3742x; 2.3742x over previous
import functools

import jax
import jax.numpy as jnp
from jax import lax
from jax.experimental import pallas as pl
from jax.experimental.pallas import tpu as pltpu
from jax.experimental.pallas import tpu_sc as plsc

_NSPECIAL = 8
_LEVEL = 2048
_SEQ = 16
_VOCAB = _NSPECIAL + (_SEQ - 1) * _LEVEL
_SMOOTH = 0.1
_CONF = 1.0 - _SMOOTH
_ROWS_PER_BLOCK = 8


def _rowstats_body(x_ref, lse_ref, msum_ref, x0_ref):
    i = pl.program_id(0)
    xb = x_ref[...]
    m = jnp.max(xb, axis=-1, keepdims=True)
    se = jnp.sum(jnp.exp(xb - m), axis=-1, keepdims=True)
    lse_ref[...] = m + jnp.log(se)
    row = lax.broadcasted_iota(jnp.int32, (_ROWS_PER_BLOCK, 1), 0)
    s = (i * _ROWS_PER_BLOCK + row) % _SEQ
    col = lax.broadcasted_iota(jnp.int32, xb.shape, 1)
    lo = _NSPECIAL + _LEVEL * s
    in_level = (s < _SEQ - 1) & (col >= lo) & (col < lo + _LEVEL)
    mask = (col < _NSPECIAL) | in_level
    msum_ref[...] = jnp.sum(jnp.where(mask, xb, 0.0), axis=-1, keepdims=True)
    x0_ref[...] = xb[:, 0:1]


def _combine_body(t_ref, xt_ref, lse_ref, msum_ref, x0_ref, out_ref):
    t = t_ref[...]
    s = lax.broadcasted_iota(jnp.int32, t.shape, 1)
    is_lvl = s < _SEQ - 1
    lo = _NSPECIAL + _LEVEL * s
    inmask = ((t < _NSPECIAL) | (is_lvl & (t >= lo) & (t < lo + _LEVEL)))
    inmask = inmask.astype(jnp.float32)
    count = jnp.where(is_lvl, float(_NSPECIAL + _LEVEL), float(_NSPECIAL))
    sm = _SMOOTH / (count - 2.0)
    xt = xt_ref[...]
    wx = _CONF * xt + sm * (msum_ref[...] - x0_ref[...] - inmask * xt)
    wtd = _CONF + sm * (count - 1.0 - inmask)
    contrib = jnp.where(t != 0, lse_ref[...] * wtd - wx, 0.0)
    out_ref[0, 0] = jnp.sum(contrib)


def _sc_gather(x_flat, t_flat):
    info = plsc.get_sparse_core_info()
    nw = info.num_cores * info.num_subcores
    lanes = info.num_lanes
    b = t_flat.shape[0]
    bpw = b // nw
    mesh = plsc.VectorSubcoreMesh(core_axis_name="c", subcore_axis_name="s")

    @functools.partial(
        pl.kernel,
        mesh=mesh,
        out_type=jax.ShapeDtypeStruct((b,), jnp.float32),
        scratch_types=[
            pltpu.VMEM((bpw,), jnp.int32),
            pltpu.VMEM((bpw,), jnp.int32),
            pltpu.VMEM((bpw,), jnp.float32),
            pltpu.SemaphoreType.DMA,
        ],
    )
    def gather_kernel(x_hbm, t_hbm, out_hbm, t_v, idx_v, vals_v, sem):
        wid = lax.axis_index("s") * info.num_cores + lax.axis_index("c")
        base = wid * bpw
        pltpu.sync_copy(t_hbm.at[pl.ds(base, bpw)], t_v)
        for c in range(bpw // lanes):
            tv = t_v[pl.ds(c * lanes, lanes)]
            rows = base + c * lanes + lax.iota(jnp.int32, lanes)
            idx_v[pl.ds(c * lanes, lanes)] = rows * _VOCAB + tv
        pltpu.async_copy(x_hbm.at[idx_v], vals_v, sem).wait()
        pltpu.sync_copy(vals_v, out_hbm.at[pl.ds(base, bpw)])

    return gather_kernel(x_flat, t_flat)


def kernel(x, target):
    batch, seq, vocab = x.shape
    n = batch * seq
    xt = _sc_gather(x.reshape(-1), target.reshape(-1))
    lse, msum, x0 = pl.pallas_call(
        _rowstats_body,
        grid=(n // _ROWS_PER_BLOCK,),
        in_specs=[pl.BlockSpec((_ROWS_PER_BLOCK, vocab), lambda i: (i, 0))],
        out_specs=[
            pl.BlockSpec((_ROWS_PER_BLOCK, 1), lambda i: (i, 0))
            for _ in range(3)
        ],
        out_shape=[jax.ShapeDtypeStruct((n, 1), jnp.float32)] * 3,
        compiler_params=pltpu.CompilerParams(
            dimension_semantics=("parallel",)
        ),
    )(x.reshape(n, vocab))
    loss = pl.pallas_call(
        _combine_body,
        in_specs=[pl.BlockSpec((batch, seq), lambda: (0, 0))] * 5,
        out_specs=pl.BlockSpec(memory_space=pltpu.SMEM),
        out_shape=jax.ShapeDtypeStruct((1, 1), jnp.float32),
    )(
        target,
        xt.reshape(batch, seq),
        lse.reshape(batch, seq),
        msum.reshape(batch, seq),
        x0.reshape(batch, seq),
    )
    return loss[0, 0]
